# baseline (speedup 1.0000x reference)
import functools

import jax
import jax.numpy as jnp
from jax import lax
from jax.experimental import pallas as pl
from jax.experimental.pallas import tpu as pltpu
from jax.experimental.pallas import tpu_sc as plsc


def _make_gather(B: int, S: int, V: int, D: int):
    NW = 32
    NS = 16
    N = B * S
    per_w = N // NW
    w_per_row = S // per_w
    CH = 8
    NBUF = 4
    MS = 4
    n_ch = per_w // CH

    mesh = plsc.VectorSubcoreMesh(core_axis_name="c", subcore_axis_name="s")

    @functools.partial(
        pl.kernel,
        mesh=mesh,
        out_type=jax.ShapeDtypeStruct((B, S, D), jnp.float32),
        scratch_types=[
            pltpu.VMEM((per_w,), jnp.int32),
            pltpu.VMEM((NBUF * CH, D), jnp.float32),
            pltpu.VMEM_SHARED((NS, MS, CH, D), jnp.float32),
            pltpu.SemaphoreType.DMA((NBUF,)),
            pltpu.SemaphoreType.DMA((MS,)),
            pltpu.SemaphoreType.DMA((MS,)),
        ],
    )
    def gather_kernel(idx_hbm, table_hbm, out_hbm,
                      idx_v, bufs, spm, gsem, msem, hsem):
        rows = [bufs.at[pl.ds(s * CH, CH)] for s in range(NBUF)]
        sid = lax.axis_index("s")
        wid = sid * 2 + lax.axis_index("c")
        b = wid // w_per_row
        col0 = (wid % w_per_row) * per_w
        pltpu.sync_copy(idx_hbm.at[b, pl.ds(col0, per_w)], idx_v)

        def start_gather(i, slot):
            pltpu.async_copy(
                table_hbm.at[idx_v.at[pl.ds(i * CH, CH)]],
                rows[slot], gsem.at[slot])

        def wait_gather(slot):
            pltpu.make_async_copy(
                table_hbm.at[pl.ds(0, CH)], rows[slot], gsem.at[slot]).wait()

        def start_move(slot, mslot):
            pltpu.async_copy(rows[slot], spm.at[sid, mslot], msem.at[mslot])

        def wait_move(slot, mslot):
            pltpu.make_async_copy(
                rows[slot], spm.at[sid, mslot], msem.at[mslot]).wait()

        def start_out(i, mslot):
            pltpu.async_copy(
                spm.at[sid, mslot],
                out_hbm.at[b, pl.ds(col0 + i * CH, CH)], hsem.at[mslot])

        def wait_out(mslot):
            pltpu.make_async_copy(
                spm.at[sid, mslot],
                out_hbm.at[b, pl.ds(col0, CH)], hsem.at[mslot]).wait()

        for s in range(NBUF - 1):
            start_gather(s, s)

        def body(k, _):
            for s in range(NBUF):
                i = k * NBUF + s
                pslot = (s - 1) % NBUF
                ms = s % MS
                pms = (s - 1) % MS
                wait_gather(s)

                @pl.when(i - MS >= 0)
                def _wh():
                    wait_out(ms)
                start_move(s, ms)

                @pl.when(i - 1 >= 0)
                def _hm():
                    wait_move(pslot, pms)
                    start_out(i - 1, pms)

                @pl.when(i + NBUF - 1 < n_ch)
                def _g():
                    start_gather(i + NBUF - 1, pslot)
            return _
        lax.fori_loop(0, n_ch // NBUF, body, 0)

        last = (n_ch - 1) % NBUF
        wait_move(last, last % MS)
        start_out(n_ch - 1, last % MS)
        for m in range(MS):
            wait_out(m)

    return gather_kernel


def kernel(input_ids, table):
    B, S = input_ids.shape
    V, D = table.shape
    return _make_gather(B, S, V, D)(input_ids, table)


# speedup vs baseline: 1.0212x; 1.0212x over previous
import functools

import jax
import jax.numpy as jnp
from jax import lax
from jax.experimental import pallas as pl
from jax.experimental.pallas import tpu as pltpu
from jax.experimental.pallas import tpu_sc as plsc


def _make_gather(B: int, S: int, V: int, D: int):
    NW = 32
    NS = 16
    N = B * S
    per_w = N // NW
    w_per_row = S // per_w
    CH = 16
    NBUF = 4
    MS = 2
    n_ch = per_w // CH

    mesh = plsc.VectorSubcoreMesh(core_axis_name="c", subcore_axis_name="s")

    @functools.partial(
        pl.kernel,
        mesh=mesh,
        out_type=jax.ShapeDtypeStruct((B, S, D), jnp.float32),
        scratch_types=[
            pltpu.VMEM((per_w,), jnp.int32),
            pltpu.VMEM((NBUF * CH, D), jnp.float32),
            pltpu.VMEM_SHARED((NS, MS, CH, D), jnp.float32),
            pltpu.SemaphoreType.DMA((NBUF,)),
            pltpu.SemaphoreType.DMA((MS,)),
            pltpu.SemaphoreType.DMA((MS,)),
        ],
    )
    def gather_kernel(idx_hbm, table_hbm, out_hbm,
                      idx_v, bufs, spm, gsem, msem, hsem):
        rows = [bufs.at[pl.ds(s * CH, CH)] for s in range(NBUF)]
        sid = lax.axis_index("s")
        wid = sid * 2 + lax.axis_index("c")
        b = wid // w_per_row
        col0 = (wid % w_per_row) * per_w
        pltpu.sync_copy(idx_hbm.at[b, pl.ds(col0, per_w)], idx_v)

        def start_gather(i, slot):
            pltpu.async_copy(
                table_hbm.at[idx_v.at[pl.ds(i * CH, CH)]],
                rows[slot], gsem.at[slot])

        def wait_gather(slot):
            pltpu.make_async_copy(
                table_hbm.at[pl.ds(0, CH)], rows[slot], gsem.at[slot]).wait()

        def start_move(slot, mslot):
            pltpu.async_copy(rows[slot], spm.at[sid, mslot], msem.at[mslot])

        def wait_move(slot, mslot):
            pltpu.make_async_copy(
                rows[slot], spm.at[sid, mslot], msem.at[mslot]).wait()

        def start_out(i, mslot):
            pltpu.async_copy(
                spm.at[sid, mslot],
                out_hbm.at[b, pl.ds(col0 + i * CH, CH)], hsem.at[mslot])

        def wait_out(mslot):
            pltpu.make_async_copy(
                spm.at[sid, mslot],
                out_hbm.at[b, pl.ds(col0, CH)], hsem.at[mslot]).wait()

        for s in range(NBUF - 1):
            start_gather(s, s)

        def body(k, _):
            for s in range(NBUF):
                i = k * NBUF + s
                pslot = (s - 1) % NBUF
                ms = s % MS
                pms = (s - 1) % MS
                wait_gather(s)

                @pl.when(i - MS >= 0)
                def _wh():
                    wait_out(ms)
                start_move(s, ms)

                @pl.when(i - 1 >= 0)
                def _hm():
                    wait_move(pslot, pms)
                    start_out(i - 1, pms)

                @pl.when(i + NBUF - 1 < n_ch)
                def _g():
                    start_gather(i + NBUF - 1, pslot)
            return _
        lax.fori_loop(0, n_ch // NBUF, body, 0)

        last = (n_ch - 1) % NBUF
        wait_move(last, last % MS)
        start_out(n_ch - 1, last % MS)
        for m in range(MS):
            wait_out(m)

    return gather_kernel


def kernel(input_ids, table):
    B, S = input_ids.shape
    V, D = table.shape
    return _make_gather(B, S, V, D)(input_ids, table)
